# baseline (speedup 1.0000x reference)
import functools

import jax
import jax.numpy as jnp
from jax import lax
from jax.experimental import pallas as pl
from jax.experimental.pallas import tpu as pltpu
from jax.experimental.pallas import tpu_sc as plsc

_B, _T, _H, _W, _C = 16, 16, 32, 32, 128
_ROW = _W * _C
_NROWS = _B * _T * _H
_NW = 32
_RPW = _NROWS // _NW
_G = 4
_NG = _RPW // _G
_NBUF = 4
_NGB = _NG // _NBUF
_GSZ = _G * _ROW
_LANES = 16
_CCHUNKS = _C // _LANES


def _body(x_hbm, t_hbm, h_hbm, w_hbm, out_hbm,
          tv, hv, wv, b0, b1, b2, b3,
          i0, i1, i2, i3, o0, o1, o2, o3):
    cid = lax.axis_index("c")
    sid = lax.axis_index("s")
    wid = sid * 2 + cid
    base = wid * _RPW
    bufs = (b0, b1, b2, b3)
    isems = (i0, i1, i2, i3)
    osems = (o0, o1, o2, o3)

    pltpu.sync_copy(t_hbm, tv)
    pltpu.sync_copy(h_hbm, hv)
    pltpu.sync_copy(w_hbm, wv)

    for b in range(2):
        start = (base + b * _G) * _ROW
        pltpu.async_copy(x_hbm.at[pl.ds(start, _GSZ)], bufs[b], isems[b])

    def outer(g0, carry):
        for s in range(_NBUF):
            g = g0 * _NBUF + s
            row0 = base + g * _G
            start = row0 * _ROW
            xb, isem, osem = bufs[s], isems[s], osems[s]

            pltpu.make_async_copy(x_hbm.at[pl.ds(start, _GSZ)], xb, isem).wait()

            threg = []
            for rho in range(_G):
                r = row0 + rho
                th = r % (_T * _H)
                t = th // _H
                h = th % _H
                tb = t * _C
                hb = h * _C
                threg.append([tv[pl.ds(tb + k * _LANES, _LANES)]
                              + hv[pl.ds(hb + k * _LANES, _LANES)]
                              for k in range(_CCHUNKS)])

            def wblk(j, c3):
                jc = j * _C
                for k in range(_CCHUNKS):
                    o = jc + k * _LANES
                    wreg = wv[pl.ds(o, _LANES)]
                    for rho in range(_G):
                        plsc.addupdate(xb.at[pl.ds(rho * _ROW + o, _LANES)],
                                       wreg + threg[rho][k])
                return c3

            lax.fori_loop(0, _W, wblk, 0)

            pltpu.async_copy(xb, out_hbm.at[pl.ds(start, _GSZ)], osem)

            s2 = (s + 2) % _NBUF
            g2 = g + 2
            if s < 2:
                @pl.when(g0 >= 1)
                def _():
                    dstart = (base + (g2 - _NBUF) * _G) * _ROW
                    pltpu.make_async_copy(
                        bufs[s2], out_hbm.at[pl.ds(dstart, _GSZ)],
                        osems[s2]).wait()

                nstart = (base + g2 * _G) * _ROW
                pltpu.async_copy(x_hbm.at[pl.ds(nstart, _GSZ)],
                                 bufs[s2], isems[s2])
            else:
                @pl.when(g0 < _NGB - 1)
                def _():
                    dstart = (base + (g2 - _NBUF) * _G) * _ROW
                    pltpu.make_async_copy(
                        bufs[s2], out_hbm.at[pl.ds(dstart, _GSZ)],
                        osems[s2]).wait()
                    nstart = (base + g2 * _G) * _ROW
                    pltpu.async_copy(x_hbm.at[pl.ds(nstart, _GSZ)],
                                     bufs[s2], isems[s2])
        return carry

    lax.fori_loop(0, _NGB, outer, 0)

    for s in range(_NBUF):
        g = _NG - _NBUF + s
        start = (base + g * _G) * _ROW
        pltpu.make_async_copy(bufs[s], out_hbm.at[pl.ds(start, _GSZ)],
                              osems[s]).wait()


@jax.jit
def _pos_embed_sc(xf, tf, hf, wf):
    mesh = plsc.VectorSubcoreMesh(core_axis_name="c", subcore_axis_name="s")
    f = functools.partial(
        pl.kernel,
        mesh=mesh,
        out_type=jax.ShapeDtypeStruct((_NROWS * _ROW,), jnp.float32),
        scratch_types=[
            pltpu.VMEM((_T * _C,), jnp.float32),
            pltpu.VMEM((_H * _C,), jnp.float32),
            pltpu.VMEM((_W * _C,), jnp.float32),
            pltpu.VMEM((_GSZ,), jnp.float32),
            pltpu.VMEM((_GSZ,), jnp.float32),
            pltpu.VMEM((_GSZ,), jnp.float32),
            pltpu.VMEM((_GSZ,), jnp.float32),
            pltpu.SemaphoreType.DMA,
            pltpu.SemaphoreType.DMA,
            pltpu.SemaphoreType.DMA,
            pltpu.SemaphoreType.DMA,
            pltpu.SemaphoreType.DMA,
            pltpu.SemaphoreType.DMA,
            pltpu.SemaphoreType.DMA,
            pltpu.SemaphoreType.DMA,
        ],
    )(_body)
    return f(xf, tf, hf, wf)


def kernel(x, T_table, H_table, W_table):
    xf = x.reshape(_NROWS * _ROW)
    out = _pos_embed_sc(xf, T_table.reshape(-1), H_table.reshape(-1),
                        W_table.reshape(-1))
    return out.reshape(x.shape)


# speedup vs baseline: 1.0224x; 1.0224x over previous
import functools

import jax
import jax.numpy as jnp
from jax import lax
from jax.experimental import pallas as pl
from jax.experimental.pallas import tpu as pltpu
from jax.experimental.pallas import tpu_sc as plsc

_B, _T, _H, _W, _C = 16, 16, 32, 32, 128
_ROW = _W * _C
_NROWS = _B * _T * _H
_NW = 32
_RPW = _NROWS // _NW
_G = 2
_NG = _RPW // _G
_NBUF = 8
_LOOK = 4
_NGB = _NG // _NBUF
_GSZ = _G * _ROW
_LANES = 16
_CCHUNKS = _C // _LANES

assert _LOOK < _NBUF and _NG % _NBUF == 0


def _body(x_hbm, t_hbm, h_hbm, w_hbm, out_hbm, tv, hv, wv, *rest):
    bufs = rest[:_NBUF]
    isems = rest[_NBUF:2 * _NBUF]
    osems = rest[2 * _NBUF:3 * _NBUF]
    cid = lax.axis_index("c")
    sid = lax.axis_index("s")
    wid = sid * 2 + cid
    base = wid * _RPW

    pltpu.sync_copy(t_hbm, tv)
    pltpu.sync_copy(h_hbm, hv)
    pltpu.sync_copy(w_hbm, wv)

    for b in range(_LOOK):
        start = (base + b * _G) * _ROW
        pltpu.async_copy(x_hbm.at[pl.ds(start, _GSZ)], bufs[b], isems[b])

    def outer(g0, carry):
        for s in range(_NBUF):
            g = g0 * _NBUF + s
            row0 = base + g * _G
            start = row0 * _ROW
            xb, isem, osem = bufs[s], isems[s], osems[s]

            pltpu.make_async_copy(x_hbm.at[pl.ds(start, _GSZ)], xb, isem).wait()

            threg = []
            for rho in range(_G):
                r = row0 + rho
                th = r % (_T * _H)
                t = th // _H
                h = th % _H
                tb = t * _C
                hb = h * _C
                threg.append([tv[pl.ds(tb + k * _LANES, _LANES)]
                              + hv[pl.ds(hb + k * _LANES, _LANES)]
                              for k in range(_CCHUNKS)])

            def wblk(j, c3):
                jc = j * _C
                for k in range(_CCHUNKS):
                    o = jc + k * _LANES
                    wreg = wv[pl.ds(o, _LANES)]
                    for rho in range(_G):
                        plsc.addupdate(xb.at[pl.ds(rho * _ROW + o, _LANES)],
                                       wreg + threg[rho][k])
                return c3

            lax.fori_loop(0, _W, wblk, 0)

            pltpu.async_copy(xb, out_hbm.at[pl.ds(start, _GSZ)], osem)

            sp = (s + _LOOK) % _NBUF
            gp = g + _LOOK
            if s + _LOOK < _NBUF:
                @pl.when(g0 >= 1)
                def _():
                    dstart = (base + (gp - _NBUF) * _G) * _ROW
                    pltpu.make_async_copy(
                        bufs[sp], out_hbm.at[pl.ds(dstart, _GSZ)],
                        osems[sp]).wait()

                nstart = (base + gp * _G) * _ROW
                pltpu.async_copy(x_hbm.at[pl.ds(nstart, _GSZ)],
                                 bufs[sp], isems[sp])
            else:
                @pl.when(g0 < _NGB - 1)
                def _():
                    dstart = (base + (gp - _NBUF) * _G) * _ROW
                    pltpu.make_async_copy(
                        bufs[sp], out_hbm.at[pl.ds(dstart, _GSZ)],
                        osems[sp]).wait()
                    nstart = (base + gp * _G) * _ROW
                    pltpu.async_copy(x_hbm.at[pl.ds(nstart, _GSZ)],
                                     bufs[sp], isems[sp])
        return carry

    lax.fori_loop(0, _NGB, outer, 0)

    for s in range(_NBUF):
        g = _NG - _NBUF + s
        start = (base + g * _G) * _ROW
        pltpu.make_async_copy(bufs[s], out_hbm.at[pl.ds(start, _GSZ)],
                              osems[s]).wait()


@jax.jit
def _pos_embed_sc(xf, tf, hf, wf):
    mesh = plsc.VectorSubcoreMesh(core_axis_name="c", subcore_axis_name="s")
    f = functools.partial(
        pl.kernel,
        mesh=mesh,
        out_type=jax.ShapeDtypeStruct((_NROWS * _ROW,), jnp.float32),
        scratch_types=(
            [pltpu.VMEM((_T * _C,), jnp.float32),
             pltpu.VMEM((_H * _C,), jnp.float32),
             pltpu.VMEM((_W * _C,), jnp.float32)]
            + [pltpu.VMEM((_GSZ,), jnp.float32) for _ in range(_NBUF)]
            + [pltpu.SemaphoreType.DMA for _ in range(2 * _NBUF)]
        ),
    )(_body)
    return f(xf, tf, hf, wf)


def kernel(x, T_table, H_table, W_table):
    xf = x.reshape(_NROWS * _ROW)
    out = _pos_embed_sc(xf, T_table.reshape(-1), H_table.reshape(-1),
                        W_table.reshape(-1))
    return out.reshape(x.shape)


# speedup vs baseline: 1.0291x; 1.0065x over previous
import functools

import jax
import jax.numpy as jnp
from jax import lax
from jax.experimental import pallas as pl
from jax.experimental.pallas import tpu as pltpu
from jax.experimental.pallas import tpu_sc as plsc

_B, _T, _H, _W, _C = 16, 16, 32, 32, 128
_ROW = _W * _C
_NROWS = _B * _T * _H
_NW = 32
_RPW = _NROWS // _NW
_G = 2
_NG = _RPW // _G
_NBUF = 8
_LOOK = 6
_NGB = _NG // _NBUF
_GSZ = _G * _ROW
_LANES = 16
_CCHUNKS = _C // _LANES

assert _LOOK < _NBUF and _NG % _NBUF == 0


def _body(x_hbm, t_hbm, h_hbm, w_hbm, out_hbm, tv, hv, wv, *rest):
    bufs = rest[:_NBUF]
    isems = rest[_NBUF:2 * _NBUF]
    osems = rest[2 * _NBUF:3 * _NBUF]
    cid = lax.axis_index("c")
    sid = lax.axis_index("s")
    wid = sid * 2 + cid
    base = wid * _RPW

    for b in range(_LOOK):
        start = (base + b * _G) * _ROW
        pltpu.async_copy(x_hbm.at[pl.ds(start, _GSZ)], bufs[b], isems[b])

    tsem = osems[0]
    ct = pltpu.async_copy(t_hbm, tv, tsem)
    ch = pltpu.async_copy(h_hbm, hv, tsem)
    cw = pltpu.async_copy(w_hbm, wv, tsem)
    ct.wait()
    ch.wait()
    cw.wait()

    def outer(g0, carry):
        for s in range(_NBUF):
            g = g0 * _NBUF + s
            row0 = base + g * _G
            start = row0 * _ROW
            xb, isem, osem = bufs[s], isems[s], osems[s]

            pltpu.make_async_copy(x_hbm.at[pl.ds(start, _GSZ)], xb, isem).wait()

            threg = []
            for rho in range(_G):
                r = row0 + rho
                th = r % (_T * _H)
                t = th // _H
                h = th % _H
                tb = t * _C
                hb = h * _C
                threg.append([tv[pl.ds(tb + k * _LANES, _LANES)]
                              + hv[pl.ds(hb + k * _LANES, _LANES)]
                              for k in range(_CCHUNKS)])

            def wblk(j, c3):
                jc = j * _C
                for k in range(_CCHUNKS):
                    o = jc + k * _LANES
                    wreg = wv[pl.ds(o, _LANES)]
                    for rho in range(_G):
                        plsc.addupdate(xb.at[pl.ds(rho * _ROW + o, _LANES)],
                                       wreg + threg[rho][k])
                return c3

            lax.fori_loop(0, _W, wblk, 0)

            pltpu.async_copy(xb, out_hbm.at[pl.ds(start, _GSZ)], osem)

            sp = (s + _LOOK) % _NBUF
            gp = g + _LOOK
            if s + _LOOK < _NBUF:
                @pl.when(g0 >= 1)
                def _():
                    dstart = (base + (gp - _NBUF) * _G) * _ROW
                    pltpu.make_async_copy(
                        bufs[sp], out_hbm.at[pl.ds(dstart, _GSZ)],
                        osems[sp]).wait()

                nstart = (base + gp * _G) * _ROW
                pltpu.async_copy(x_hbm.at[pl.ds(nstart, _GSZ)],
                                 bufs[sp], isems[sp])
            else:
                @pl.when(g0 < _NGB - 1)
                def _():
                    dstart = (base + (gp - _NBUF) * _G) * _ROW
                    pltpu.make_async_copy(
                        bufs[sp], out_hbm.at[pl.ds(dstart, _GSZ)],
                        osems[sp]).wait()
                    nstart = (base + gp * _G) * _ROW
                    pltpu.async_copy(x_hbm.at[pl.ds(nstart, _GSZ)],
                                     bufs[sp], isems[sp])
        return carry

    lax.fori_loop(0, _NGB, outer, 0)

    for s in range(_NBUF):
        g = _NG - _NBUF + s
        start = (base + g * _G) * _ROW
        pltpu.make_async_copy(bufs[s], out_hbm.at[pl.ds(start, _GSZ)],
                              osems[s]).wait()


@jax.jit
def _pos_embed_sc(xf, tf, hf, wf):
    mesh = plsc.VectorSubcoreMesh(core_axis_name="c", subcore_axis_name="s")
    f = functools.partial(
        pl.kernel,
        mesh=mesh,
        out_type=jax.ShapeDtypeStruct((_NROWS * _ROW,), jnp.float32),
        scratch_types=(
            [pltpu.VMEM((_T * _C,), jnp.float32),
             pltpu.VMEM((_H * _C,), jnp.float32),
             pltpu.VMEM((_W * _C,), jnp.float32)]
            + [pltpu.VMEM((_GSZ,), jnp.float32) for _ in range(_NBUF)]
            + [pltpu.SemaphoreType.DMA for _ in range(2 * _NBUF)]
        ),
    )(_body)
    return f(xf, tf, hf, wf)


def kernel(x, T_table, H_table, W_table):
    xf = x.reshape(_NROWS * _ROW)
    out = _pos_embed_sc(xf, T_table.reshape(-1), H_table.reshape(-1),
                        W_table.reshape(-1))
    return out.reshape(x.shape)
